# Initial kernel scaffold; baseline (speedup 1.0000x reference)
import jax
import jax.numpy as jnp
from jax.experimental import pallas as pl


def kernel(x, edge_index, W, attn_j, attn_i, bias):
    raise NotImplementedError("write your pallas kernel here")


# speedup vs baseline: 11.2060x; 11.2060x over previous
import jax
import jax.numpy as jnp
from jax import lax
from jax.experimental import pallas as pl
from jax.experimental.pallas import tpu as pltpu
from jax.experimental.pallas import tpu_sc as plsc

N = 10000
E = 320000
D = 128
NEG = 0.2
NPAD = 10240
TRASH = 10200
NC, NS, L = 2, 16, 16
EPT = E // (NC * NS)
CH = 80
NCHUNK = EPT // CH
ROWS_T = NPAD // NS
W1 = 16000

_SC_PARAMS = pltpu.CompilerParams(needs_layout_passes=False,
                                  use_tc_tiling_on_sc=False)


def _scmesh():
    return plsc.VectorSubcoreMesh(core_axis_name="c", subcore_axis_name="s",
                                  num_cores=NC, num_subcores=NS)


def _tc_prep_body(x_ref, w_ref, v_ref, er_ref, xw_ref, aj_ref, ai_ref,
                  cnt_ref):
    xw = lax.dot_general(x_ref[...], w_ref[...],
                         (((1,), (1,)), ((), ())),
                         preferred_element_type=jnp.float32)
    xw_ref[...] = xw
    sc = lax.dot_general(xw, v_ref[...],
                         (((1,), (0,)), ((), ())),
                         preferred_element_type=jnp.float32)
    aj_ref[...] = sc[:, 0:1]
    ai_ref[...] = sc[:, 1:2]
    valid = (er_ref[0] != er_ref[1]).astype(jnp.int32)
    cnt_ref[...] = jnp.sum(valid, axis=1, keepdims=True)


def _tc_prep(x, W, V, er):
    return pl.pallas_call(
        _tc_prep_body,
        out_shape=(
            jax.ShapeDtypeStruct((N, D), jnp.float32),
            jax.ShapeDtypeStruct((N, 1), jnp.float32),
            jax.ShapeDtypeStruct((N, 1), jnp.float32),
            jax.ShapeDtypeStruct((32, 1), jnp.int32),
        ),
    )(x, W, V, er)


def _sc_pre_body(src_hbm, dst_hbm, aj_hbm, ai_hbm, cnt_hbm,
                 bj_hbm, bi_hbm, es_hbm,
                 cnt_v, srcw, dstw, schk, dchk, aj_v, ai_v,
                 J1_v, I1_v, bj_v, bi_v, es_v):
    c = lax.axis_index("c")
    s = lax.axis_index("s")
    i16 = lax.iota(jnp.int32, L)

    @pl.when((c == 0) & (s == 0))
    def _only():
        pltpu.sync_copy(cnt_hbm, cnt_v)
        c0 = cnt_v[pl.ds(0, L)]
        c1 = cnt_v[pl.ds(L, L)]
        e_valid = jnp.sum(c0, axis=0) + jnp.sum(c1, axis=0)

        def initb(g, carry):
            v = g * L + i16 - e_valid
            J1_v[pl.ds(g * L, L)] = v
            I1_v[pl.ds(g * L, L)] = v
            return carry

        lax.fori_loop(0, NPAD // L, initb, 0)

        def scan_group(sv, dv, pos):
            m32 = (sv != dv).astype(jnp.int32)
            kvec = plsc.cumsum(m32) + (pos - 1)
            qual = (m32 > 0) & (kvec < N)
            plsc.store_scatter(J1_v, [kvec], sv, mask=qual)
            plsc.store_scatter(I1_v, [kvec], dv, mask=qual)
            return pos + jnp.sum(m32, axis=0)

        pltpu.sync_copy(src_hbm.at[pl.ds(0, W1)], srcw)
        pltpu.sync_copy(dst_hbm.at[pl.ds(0, W1)], dstw)

        def cond_a(carry):
            g, pos = carry
            return (pos < N) & (g < W1 // L)

        def body_a(carry):
            g, pos = carry
            sv = srcw[pl.ds(g * L, L)]
            dv = dstw[pl.ds(g * L, L)]
            return g + 1, scan_group(sv, dv, pos)

        _, pos = lax.while_loop(cond_a, body_a,
                                (jnp.int32(0), jnp.int32(0)))

        def cond_b(carry):
            ch, pos = carry
            return (pos < N) & (ch < E // CH)

        def body_b(carry):
            ch, pos = carry
            pltpu.sync_copy(src_hbm.at[pl.ds(ch * CH, CH)], schk)
            pltpu.sync_copy(dst_hbm.at[pl.ds(ch * CH, CH)], dchk)
            for g in range(CH // L):
                sv = schk[pl.ds(g * L, L)]
                dv = dchk[pl.ds(g * L, L)]
                pos = scan_group(sv, dv, pos)
            return ch + 1, pos

        lax.while_loop(cond_b, body_b, (jnp.int32(W1 // CH), pos))

        pltpu.sync_copy(aj_hbm, aj_v)
        pltpu.sync_copy(ai_hbm, ai_v)

        def gb(g, carry):
            j16 = jnp.clip(J1_v[pl.ds(g * L, L)], 0, N - 1)
            i16c = jnp.clip(I1_v[pl.ds(g * L, L)], 0, N - 1)
            bjv = plsc.load_gather(aj_v, [j16])
            biv = plsc.load_gather(ai_v, [i16c])
            z = bjv + biv
            bj_v[pl.ds(g * L, L)] = bjv
            bi_v[pl.ds(g * L, L)] = biv
            es_v[pl.ds(g * L, L)] = jnp.exp(jnp.maximum(z, NEG * z))
            return carry

        lax.fori_loop(0, NPAD // L, gb, 0)
        pltpu.sync_copy(bj_v, bj_hbm)
        pltpu.sync_copy(bi_v, bi_hbm)
        pltpu.sync_copy(es_v, es_hbm)


def _sc_pre(src, dst, aj, ai, counts):
    f32, i32 = jnp.float32, jnp.int32
    kern = pl.kernel(
        _sc_pre_body,
        out_type=(
            jax.ShapeDtypeStruct((NPAD,), f32),
            jax.ShapeDtypeStruct((NPAD,), f32),
            jax.ShapeDtypeStruct((NPAD,), f32),
        ),
        mesh=_scmesh(),
        compiler_params=_SC_PARAMS,
        scratch_types=[
            pltpu.VMEM((32,), i32),
            pltpu.VMEM((W1,), i32),
            pltpu.VMEM((W1,), i32),
            pltpu.VMEM((CH,), i32),
            pltpu.VMEM((CH,), i32),
            pltpu.VMEM((N,), f32),
            pltpu.VMEM((N,), f32),
            pltpu.VMEM((NPAD,), i32),
            pltpu.VMEM((NPAD,), i32),
            pltpu.VMEM((NPAD,), f32),
            pltpu.VMEM((NPAD,), f32),
            pltpu.VMEM((NPAD,), f32),
        ],
    )
    return kern(src, dst, aj, ai, counts)


def _sc_edge_body(src_hbm, dst_hbm, bj_hbm, bi_hbm, xw_hbm,
                  P_hbm, S_hbm,
                  bj_v, bi_v, rows_v, sbuf, dbuf, sidx_v, expa_v,
                  P_sh, S_sh, sem_s, sem_d, sem_g):
    c = lax.axis_index("c")
    s = lax.axis_index("s")
    i16 = lax.iota(jnp.int32, L)
    zero16 = jnp.zeros((L,), jnp.int32)
    z16 = jnp.zeros((L,), jnp.float32)
    wid = c * NS + s
    ebase = wid * EPT

    pltpu.async_copy(src_hbm.at[pl.ds(ebase, CH)], sbuf.at[0], sem_s)
    pltpu.async_copy(dst_hbm.at[pl.ds(ebase, CH)], dbuf.at[0], sem_d)

    pltpu.sync_copy(bj_hbm, bj_v)
    pltpu.sync_copy(bi_hbm, bi_v)

    def zex(r, carry):
        expa_v[r, pl.ds(0, L)] = z16
        return carry

    lax.fori_loop(0, CH, zex, 0)

    def zrows(r, carry):
        for d in range(D // L):
            rows_v[r, pl.ds(d * L, L)] = z16
        return carry

    lax.fori_loop(0, CH, zrows, 0)
    for p in range(ROWS_T // CH):
        pltpu.sync_copy(rows_v, P_sh.at[pl.ds(s * ROWS_T + p * CH, CH)])
        pltpu.sync_copy(expa_v, S_sh.at[pl.ds(s * ROWS_T + p * CH, CH)])
    plsc.subcore_barrier()

    def body(ch, carry):
        b = lax.rem(ch, 2)
        pltpu.make_async_copy(src_hbm.at[pl.ds(ebase + ch * CH, CH)],
                              sbuf.at[b], sem_s).wait()
        pltpu.make_async_copy(dst_hbm.at[pl.ds(ebase + ch * CH, CH)],
                              dbuf.at[b], sem_d).wait()

        @pl.when(ch + 1 < NCHUNK)
        def _prefetch():
            nb = lax.rem(ch + 1, 2)
            pltpu.async_copy(src_hbm.at[pl.ds(ebase + (ch + 1) * CH, CH)],
                             sbuf.at[nb], sem_s)
            pltpu.async_copy(dst_hbm.at[pl.ds(ebase + (ch + 1) * CH, CH)],
                             dbuf.at[nb], sem_d)

        cp = pltpu.async_copy(xw_hbm.at[sbuf.at[b]], rows_v, sem_g)

        for g in range(CH // L):
            sv = sbuf[b, pl.ds(g * L, L)]
            dv = dbuf[b, pl.ds(g * L, L)]
            bjv = plsc.load_gather(bj_v, [sv])
            biv = plsc.load_gather(bi_v, [dv])
            z = bjv + biv
            ex = jnp.exp(jnp.maximum(z, NEG * z))
            plsc.store_scatter(expa_v, [g * L + i16, zero16], ex)
            sidx_v[pl.ds(g * L, L)] = jnp.where(sv != dv, dv, TRASH)
        cp.wait()

        def scale(e, carry2):
            cf = plsc.load_gather(expa_v, [jnp.full((L,), e, jnp.int32),
                                           zero16])
            for d in range(D // L):
                rows_v[e, pl.ds(d * L, L)] = rows_v[e, pl.ds(d * L, L)] * cf
            return carry2

        lax.fori_loop(0, CH, scale, 0)
        pltpu.sync_copy(rows_v, P_sh.at[sidx_v], add=True)
        pltpu.sync_copy(expa_v, S_sh.at[sidx_v], add=True)
        return carry

    lax.fori_loop(0, NCHUNK, body, 0)
    plsc.subcore_barrier()
    pltpu.sync_copy(P_sh.at[pl.ds(s * ROWS_T, ROWS_T)],
                    P_hbm.at[c, pl.ds(s * ROWS_T, ROWS_T)])
    pltpu.sync_copy(S_sh.at[pl.ds(s * ROWS_T, ROWS_T)],
                    S_hbm.at[c, pl.ds(s * ROWS_T, ROWS_T)])


def _sc_edge(src, dst, bj, bi, xw):
    f32, i32 = jnp.float32, jnp.int32
    kern = pl.kernel(
        _sc_edge_body,
        out_type=(
            jax.ShapeDtypeStruct((NC, NPAD, D), f32),
            jax.ShapeDtypeStruct((NC, NPAD, L), f32),
        ),
        mesh=_scmesh(),
        compiler_params=_SC_PARAMS,
        scratch_types=[
            pltpu.VMEM((NPAD,), f32),
            pltpu.VMEM((NPAD,), f32),
            pltpu.VMEM((CH, D), f32),
            pltpu.VMEM((2, CH), i32),
            pltpu.VMEM((2, CH), i32),
            pltpu.VMEM((CH,), i32),
            pltpu.VMEM((CH, L), f32),
            pltpu.VMEM_SHARED((NPAD, D), f32),
            pltpu.VMEM_SHARED((NPAD, L), f32),
            pltpu.SemaphoreType.DMA,
            pltpu.SemaphoreType.DMA,
            pltpu.SemaphoreType.DMA,
        ],
    )
    return kern(src, dst, bj, bi, xw)


def _tc_comb_body(p_ref, s_ref, es_ref, xw_ref, b_ref, o_ref):
    esv = es_ref[...]
    num = p_ref[0] + p_ref[1] + esv * xw_ref[...]
    den = s_ref[0, :, 0:1] + s_ref[1, :, 0:1] + esv
    o_ref[...] = num / den + b_ref[...]


def _tc_comb(P, S, es, xw, bias):
    blk = 2000
    grid = N // blk
    return pl.pallas_call(
        _tc_comb_body,
        grid=(grid,),
        in_specs=[
            pl.BlockSpec((NC, blk, D), lambda i: (0, i, 0)),
            pl.BlockSpec((NC, blk, L), lambda i: (0, i, 0)),
            pl.BlockSpec((blk, 1), lambda i: (i, 0)),
            pl.BlockSpec((blk, D), lambda i: (i, 0)),
            pl.BlockSpec((1, D), lambda i: (0, 0)),
        ],
        out_specs=pl.BlockSpec((blk, D), lambda i: (i, 0)),
        out_shape=jax.ShapeDtypeStruct((N, D), jnp.float32),
    )(P, S, es, xw, bias)


def kernel(x, edge_index, W, attn_j, attn_i, bias):
    vj = attn_j.reshape(D)
    vi = attn_i.reshape(D)
    V = jnp.stack([vj, vi], axis=1)
    er = edge_index.reshape(2, 32, E // 32)
    src = edge_index[0]
    dst = edge_index[1]

    xw, aj2, ai2, cnt = _tc_prep(x, W, V, er)
    bj, bi, es = _sc_pre(src, dst, aj2.reshape(N), ai2.reshape(N),
                         cnt.reshape(32))
    P, S = _sc_edge(src, dst, bj, bi, xw)
    out = _tc_comb(P, S, es[:N].reshape(N, 1), xw, bias.reshape(1, D))
    return out
